# baseline (speedup 1.0000x reference)
import functools

import jax
import jax.numpy as jnp
from jax import lax
from jax.experimental import pallas as pl
from jax.experimental.pallas import tpu as pltpu
from jax.experimental.pallas import tpu_sc as plsc

VOCAB = 100000
NF = 26
ED = 106
B = 16384
H = 10

NC = 2
NS = 16
L = 16
NW = NC * NS

BNF = B * NF
R = BNF // NW
C = 128
NCHUNK = R // C


def _sc_gather(idx_flat, offs, tables_flat):
    mesh = plsc.VectorSubcoreMesh(core_axis_name="c", subcore_axis_name="s")

    @functools.partial(
        pl.kernel,
        out_type=jax.ShapeDtypeStruct((BNF, ED), jnp.float32),
        mesh=mesh,
        scratch_types=[
            pltpu.VMEM((R,), jnp.int32),
            pltpu.VMEM((R,), jnp.int32),
            pltpu.VMEM((C, ED), jnp.float32),
            pltpu.VMEM((C, ED), jnp.float32),
            pltpu.SemaphoreType.DMA,
            pltpu.SemaphoreType.DMA,
        ],
        compiler_params=pltpu.CompilerParams(use_tc_tiling_on_sc=False),
    )
    def gather_kernel(idx_hbm, offs_hbm, tab_hbm, out_hbm,
                      idx_v, offs_v, buf0, buf1, gsem, wsem):
        wid = lax.axis_index("c") * NS + lax.axis_index("s")
        base = wid * R

        pltpu.sync_copy(idx_hbm.at[pl.ds(base, R)], idx_v)
        pltpu.sync_copy(offs_hbm, offs_v)

        def add_body(i, _):
            sl = pl.ds(i * L, L)
            idx_v[sl] = idx_v[sl] + offs_v[sl]
            return 0
        lax.fori_loop(0, R // L, add_body, 0, unroll=4)

        bufs = (buf0, buf1)

        def g_start(c, buf):
            pltpu.async_copy(tab_hbm.at[idx_v.at[pl.ds(c * C, C)]], buf, gsem)

        def g_wait(c, buf):
            pltpu.make_async_copy(
                tab_hbm.at[idx_v.at[pl.ds(c * C, C)]], buf, gsem).wait()

        def w_start(c, buf):
            pltpu.async_copy(buf, out_hbm.at[pl.ds(base + c * C, C)], wsem)

        def w_wait(c, buf):
            pltpu.make_async_copy(
                buf, out_hbm.at[pl.ds(base + c * C, C)], wsem).wait()

        g_start(0, buf0)
        g_start(1, buf1)

        def loop_body(i, _):
            for k in range(2):
                c = 2 * i + k
                buf = bufs[k]
                g_wait(c, buf)
                w_start(c, buf)
                w_wait(c, buf)

                @pl.when(c + 2 < NCHUNK)
                def _():
                    g_start(c + 2, buf)
            return 0
        lax.fori_loop(0, NCHUNK // 2, loop_body, 0)

    return gather_kernel(idx_flat, offs, tables_flat)


def _tc_mlp(emb, W1, b1, W2, b2, W3, b3):
    BM = 1024
    D = NF * ED

    def mlp_kernel(x_ref, w1_ref, b1_ref, w2_ref, b2_ref, w3_ref, b3_ref,
                   o_ref):
        x = x_ref[...]
        h1 = lax.dot_general(x, w1_ref[...], (((1,), (1,)), ((), ())),
                             preferred_element_type=jnp.float32)
        h1 = jnp.maximum(h1 + b1_ref[...][None, :], 0.0)
        h2 = lax.dot_general(h1, w2_ref[...], (((1,), (1,)), ((), ())),
                             preferred_element_type=jnp.float32)
        h2 = jnp.maximum(h2 + b2_ref[...][None, :], 0.0)
        o = lax.dot_general(h2, w3_ref[...], (((1,), (1,)), ((), ())),
                            preferred_element_type=jnp.float32)
        o_ref[...] = jax.nn.sigmoid(o[:, 0] + b3_ref[0])

    return pl.pallas_call(
        mlp_kernel,
        grid=(B // BM,),
        in_specs=[
            pl.BlockSpec((BM, D), lambda i: (i, 0)),
            pl.BlockSpec((H, D), lambda i: (0, 0)),
            pl.BlockSpec((H,), lambda i: (0,)),
            pl.BlockSpec((H, H), lambda i: (0, 0)),
            pl.BlockSpec((H,), lambda i: (0,)),
            pl.BlockSpec((1, H), lambda i: (0, 0)),
            pl.BlockSpec((1,), lambda i: (0,)),
        ],
        out_specs=pl.BlockSpec((BM,), lambda i: (i,)),
        out_shape=jax.ShapeDtypeStruct((B,), jnp.float32),
    )(emb, W1, b1, W2, b2, W3, b3)


def kernel(sparse_feature, tables, W1, b1, W2, b2, W3, b3):
    idx_flat = sparse_feature.astype(jnp.int32).reshape(-1)
    offs = jnp.tile(jnp.arange(NF, dtype=jnp.int32) * VOCAB, R // NF)
    tables_flat = tables.reshape(NF * VOCAB, ED)
    emb = _sc_gather(idx_flat, offs, tables_flat)
    return _tc_mlp(emb.reshape(B, NF * ED), W1, b1, W2, b2, W3, b3)


# speedup vs baseline: 1.5187x; 1.5187x over previous
import functools

import jax
import jax.numpy as jnp
from jax import lax
from jax.experimental import pallas as pl
from jax.experimental.pallas import tpu as pltpu
from jax.experimental.pallas import tpu_sc as plsc

VOCAB = 100000
NF = 26
ED = 106
B = 16384
H = 10

NC = 2
NS = 16
L = 16
NW = NC * NS

BNF = B * NF
R = BNF // NW
C = 128
NCHUNK = R // C


def _sc_gather(idx_flat, offs, tables_flat):
    mesh = plsc.VectorSubcoreMesh(core_axis_name="c", subcore_axis_name="s")

    @functools.partial(
        pl.kernel,
        out_type=jax.ShapeDtypeStruct((BNF, ED), jnp.float32),
        mesh=mesh,
        scratch_types=[
            pltpu.VMEM((R,), jnp.int32),
            pltpu.VMEM((R,), jnp.int32),
            pltpu.VMEM((C, ED), jnp.float32),
            pltpu.VMEM((C, ED), jnp.float32),
            pltpu.SemaphoreType.DMA,
            pltpu.SemaphoreType.DMA,
        ],
    )
    def gather_kernel(idx_hbm, offs_hbm, tab_hbm, out_hbm,
                      idx_v, offs_v, buf0, buf1, gsem, wsem):
        wid = lax.axis_index("c") * NS + lax.axis_index("s")
        base = wid * R

        pltpu.sync_copy(idx_hbm.at[pl.ds(base, R)], idx_v)
        pltpu.sync_copy(offs_hbm, offs_v)

        def add_body(i, _):
            sl = pl.ds(i * L, L)
            idx_v[sl] = idx_v[sl] + offs_v[sl]
            return 0
        lax.fori_loop(0, R // L, add_body, 0, unroll=8)

        bufs = (buf0, buf1)

        def g_start(c, buf):
            for g in range(C // L):
                xv = idx_v[pl.ds(c * C + g * L, L)]
                for k in range(L):
                    pltpu.async_copy(
                        tab_hbm.at[pl.ds(xv[k], 1)],
                        buf.at[pl.ds(g * L + k, 1)],
                        gsem)

        def g_wait(buf):
            pltpu.make_async_copy(
                tab_hbm.at[pl.ds(0, C)], buf, gsem).wait()

        def w_start(c, buf):
            pltpu.async_copy(buf, out_hbm.at[pl.ds(base + c * C, C)], wsem)

        def w_wait(c, buf):
            pltpu.make_async_copy(
                buf, out_hbm.at[pl.ds(base + c * C, C)], wsem).wait()

        g_start(0, buf0)
        g_start(1, buf1)

        def loop_body(i, _):
            for k in range(2):
                c = 2 * i + k
                buf = bufs[k]
                g_wait(buf)
                w_start(c, buf)
                w_wait(c, buf)

                @pl.when(c + 2 < NCHUNK)
                def _():
                    g_start(c + 2, buf)
            return 0
        lax.fori_loop(0, NCHUNK // 2, loop_body, 0)

    return gather_kernel(idx_flat, offs, tables_flat)


def _tc_mlp(emb, W1, b1, W2, b2, W3, b3):
    BM = 1024
    D = NF * ED

    def mlp_kernel(x_ref, w1_ref, b1_ref, w2_ref, b2_ref, w3_ref, b3_ref,
                   o_ref):
        x = x_ref[...]
        h1 = lax.dot_general(x, w1_ref[...], (((1,), (1,)), ((), ())),
                             preferred_element_type=jnp.float32)
        h1 = jnp.maximum(h1 + b1_ref[...][None, :], 0.0)
        h2 = lax.dot_general(h1, w2_ref[...], (((1,), (1,)), ((), ())),
                             preferred_element_type=jnp.float32)
        h2 = jnp.maximum(h2 + b2_ref[...][None, :], 0.0)
        o = lax.dot_general(h2, w3_ref[...], (((1,), (1,)), ((), ())),
                            preferred_element_type=jnp.float32)
        o_ref[...] = jax.nn.sigmoid(o[:, 0] + b3_ref[0])

    return pl.pallas_call(
        mlp_kernel,
        grid=(B // BM,),
        in_specs=[
            pl.BlockSpec((BM, D), lambda i: (i, 0)),
            pl.BlockSpec((H, D), lambda i: (0, 0)),
            pl.BlockSpec((H,), lambda i: (0,)),
            pl.BlockSpec((H, H), lambda i: (0, 0)),
            pl.BlockSpec((H,), lambda i: (0,)),
            pl.BlockSpec((1, H), lambda i: (0, 0)),
            pl.BlockSpec((1,), lambda i: (0,)),
        ],
        out_specs=pl.BlockSpec((BM,), lambda i: (i,)),
        out_shape=jax.ShapeDtypeStruct((B,), jnp.float32),
    )(emb, W1, b1, W2, b2, W3, b3)


def kernel(sparse_feature, tables, W1, b1, W2, b2, W3, b3):
    idx_flat = sparse_feature.astype(jnp.int32).reshape(-1)
    offs = jnp.tile(jnp.arange(NF, dtype=jnp.int32) * VOCAB, R // NF)
    tables_flat = tables.reshape(NF * VOCAB, ED)
    emb = _sc_gather(idx_flat, offs, tables_flat)
    return _tc_mlp(emb.reshape(B, NF * ED), W1, b1, W2, b2, W3, b3)


# speedup vs baseline: 7.3392x; 4.8324x over previous
import functools

import jax
import jax.numpy as jnp
from jax import lax
from jax.experimental import pallas as pl
from jax.experimental.pallas import tpu as pltpu
from jax.experimental.pallas import tpu_sc as plsc

VOCAB = 100000
NF = 26
ED = 106
B = 16384
H = 10
HP = 16

NC = 2
NS = 16
L = 16
NW = NC * NS

BW = B // NW
C = 128
CPF = BW // C
VB = 12800


def _tc_project(tt, W1p):
    def proj_kernel(w_ref, t_ref, p_ref):
        t = t_ref[0]
        w = w_ref[0]
        p_ref[0] = lax.dot_general(t, w, (((0,), (0,)), ((), ())),
                                   preferred_element_type=jnp.float32)

    nv = (VOCAB + VB - 1) // VB
    return pl.pallas_call(
        proj_kernel,
        grid=(NF, nv),
        in_specs=[
            pl.BlockSpec((1, ED, HP), lambda f, j: (f, 0, 0)),
            pl.BlockSpec((1, ED, VB), lambda f, j: (f, 0, j)),
        ],
        out_specs=pl.BlockSpec((1, VB, HP), lambda f, j: (f, j, 0)),
        out_shape=jax.ShapeDtypeStruct((NF, VOCAB, HP), jnp.float32),
    )(W1p, tt)


def _sc_gather(idxT, P):
    mesh = plsc.VectorSubcoreMesh(core_axis_name="c", subcore_axis_name="s")

    @functools.partial(
        pl.kernel,
        out_type=jax.ShapeDtypeStruct((NF * B, HP), jnp.float32),
        mesh=mesh,
        scratch_types=[
            pltpu.VMEM((NF, BW), jnp.int32),
            pltpu.VMEM((C, HP), jnp.float32),
            pltpu.VMEM((C, HP), jnp.float32),
            pltpu.SemaphoreType.DMA,
            pltpu.SemaphoreType.DMA,
        ],
    )
    def gather_kernel(idx_hbm, p_hbm, out_hbm, idx_v, buf0, buf1, gsem, wsem):
        wid = lax.axis_index("c") * NS + lax.axis_index("s")
        b0 = wid * BW

        pltpu.sync_copy(idx_hbm.at[:, pl.ds(b0, BW)], idx_v)

        bufs = (buf0, buf1)

        def g_start(t, buf):
            f = t // CPF
            c = lax.rem(t, CPF)
            for g in range(C // L):
                xv = idx_v[f, pl.ds(c * C + g * L, L)] + f * VOCAB
                for k in range(L):
                    pltpu.async_copy(
                        p_hbm.at[pl.ds(xv[k], 1)],
                        buf.at[pl.ds(g * L + k, 1)],
                        gsem)

        def g_wait(buf):
            pltpu.make_async_copy(p_hbm.at[pl.ds(0, C)], buf, gsem).wait()

        def w_row0(t):
            f = t // CPF
            c = lax.rem(t, CPF)
            return f * B + b0 + c * C

        def w_start(t, buf):
            pltpu.async_copy(buf, out_hbm.at[pl.ds(w_row0(t), C)], wsem)

        def w_wait(t, buf):
            pltpu.make_async_copy(
                buf, out_hbm.at[pl.ds(w_row0(t), C)], wsem).wait()

        NT = NF * CPF
        g_start(0, buf0)
        g_start(1, buf1)

        def loop_body(i, _):
            for k in range(2):
                t = 2 * i + k
                buf = bufs[k]
                g_wait(buf)
                w_start(t, buf)
                w_wait(t, buf)

                @pl.when(t + 2 < NT)
                def _():
                    g_start(t + 2, buf)
            return 0
        lax.fori_loop(0, NT // 2, loop_body, 0)

    return gather_kernel(idxT, P)


def _tc_mlp(g, b1p, W2p, b2p, W3p, b3):
    BM = 1024

    def mlp_kernel(g_ref, b1_ref, w2_ref, b2_ref, w3_ref, b3_ref, o_ref):
        acc = g_ref[0]
        for f in range(1, NF):
            acc = acc + g_ref[f]
        d1 = jnp.maximum(acc + b1_ref[...][None, :], 0.0)
        h2 = lax.dot_general(d1, w2_ref[...], (((1,), (1,)), ((), ())),
                             preferred_element_type=jnp.float32)
        h2 = jnp.maximum(h2 + b2_ref[...][None, :], 0.0)
        o = lax.dot_general(h2, w3_ref[...], (((1,), (1,)), ((), ())),
                            preferred_element_type=jnp.float32)
        o_ref[...] = jax.nn.sigmoid(o[:, 0] + b3_ref[0])

    return pl.pallas_call(
        mlp_kernel,
        grid=(B // BM,),
        in_specs=[
            pl.BlockSpec((NF, BM, HP), lambda i: (0, i, 0)),
            pl.BlockSpec((HP,), lambda i: (0,)),
            pl.BlockSpec((HP, HP), lambda i: (0, 0)),
            pl.BlockSpec((HP,), lambda i: (0,)),
            pl.BlockSpec((1, HP), lambda i: (0, 0)),
            pl.BlockSpec((1,), lambda i: (0,)),
        ],
        out_specs=pl.BlockSpec((BM,), lambda i: (i,)),
        out_shape=jax.ShapeDtypeStruct((B,), jnp.float32),
    )(g.reshape(NF, B, HP), b1p, W2p, b2p, W3p, b3)


def kernel(sparse_feature, tables, W1, b1, W2, b2, W3, b3):
    tt = jnp.transpose(tables, (0, 2, 1))
    idxT = sparse_feature.astype(jnp.int32).T
    W1p = jnp.pad(jnp.transpose(W1.reshape(H, NF, ED), (1, 2, 0)),
                  ((0, 0), (0, 0), (0, HP - H)))
    b1p = jnp.pad(b1, (0, HP - H))
    b2p = jnp.pad(b2, (0, HP - H))
    W2p = jnp.pad(W2, ((0, HP - H), (0, HP - H)))
    W3p = jnp.pad(W3, ((0, 0), (0, HP - H)))
    P = _tc_project(tt, W1p).reshape(NF * VOCAB, HP)
    g = _sc_gather(idxT, P)
    return _tc_mlp(g, b1p, W2p, b2p, W3p, b3)
